# baseline (speedup 1.0000x reference)
import functools

import jax
import jax.numpy as jnp
from jax import lax
from jax.experimental import pallas as pl
from jax.experimental.pallas import tpu as pltpu
from jax.experimental.pallas import tpu_sc as plsc

_VOCAB = 100000
_EMB_DIM = 200
_CTX = 2
_HIDDEN = 128

_TILE = 2048
_NT = (_VOCAB + _TILE - 1) // _TILE

_TILE2 = 8192
_NT2 = (_VOCAB + _TILE2 - 1) // _TILE2


def _sc_gather(emb, idx):
    mesh = plsc.VectorSubcoreMesh(core_axis_name="c", subcore_axis_name="s")

    @functools.partial(
        pl.kernel,
        mesh=mesh,
        out_type=jax.ShapeDtypeStruct((_CTX, _EMB_DIM), jnp.float32),
        scratch_types=[
            pltpu.VMEM((16,), jnp.int32),
            pltpu.VMEM((_CTX, _EMB_DIM), jnp.float32),
        ],
    )
    def k(emb_hbm, idx_hbm, out_hbm, idx_v, rows_v):
        wid = lax.axis_index("s") * 2 + lax.axis_index("c")

        @pl.when(wid == 0)
        def _():
            pltpu.sync_copy(idx_hbm, idx_v.at[pl.ds(0, _CTX)])
            lanes = idx_v[...]
            for r in range(_CTX):
                row = lanes[r]
                pltpu.sync_copy(emb_hbm.at[pl.ds(row, 1), :],
                                rows_v.at[pl.ds(r, 1), :])
            pltpu.sync_copy(rows_v, out_hbm)

    return k(emb, idx)


def _k1_body(emb_ref, w1_ref, b1_ref, w2_ref, b2_ref,
             out_ref, lse_ref, h_ref, m_ref, s_ref):
    i = pl.program_id(0)

    @pl.when(i == 0)
    def _():
        h = jnp.dot(emb_ref[...], w1_ref[...],
                    preferred_element_type=jnp.float32) + b1_ref[...]
        h_ref[...] = jnp.maximum(h, 0.0)
        m_ref[...] = jnp.full((1, 1), -1e30, jnp.float32)
        s_ref[...] = jnp.zeros((1, 1), jnp.float32)

    t = jnp.dot(h_ref[...], w2_ref[...],
                preferred_element_type=jnp.float32) + b2_ref[...]
    out_ref[...] = t

    col = lax.broadcasted_iota(jnp.int32, (1, _TILE), 1) + i * _TILE
    tm = jnp.where(col < _VOCAB, t, -1e30)
    tile_max = jnp.max(tm)
    m_old = m_ref[...]
    m_new = jnp.maximum(m_old, tile_max)
    s_new = s_ref[...] * jnp.exp(m_old - m_new) + jnp.sum(jnp.exp(tm - m_new))
    m_ref[...] = m_new
    s_ref[...] = s_new

    @pl.when(i == _NT - 1)
    def _():
        lse_ref[...] = jnp.broadcast_to(m_new + jnp.log(s_new), (1, 128))


def _k2_body(x_ref, lse_ref, o_ref):
    o_ref[...] = x_ref[...] - lse_ref[:, :1]


def _mlp(embeds, W1, b1, W2, b2, interpret=False):
    out, lse = pl.pallas_call(
        _k1_body,
        grid=(_NT,),
        in_specs=[
            pl.BlockSpec((1, _CTX * _EMB_DIM), lambda i: (0, 0)),
            pl.BlockSpec((_CTX * _EMB_DIM, _HIDDEN), lambda i: (0, 0)),
            pl.BlockSpec((1, _HIDDEN), lambda i: (0, 0)),
            pl.BlockSpec((_HIDDEN, _TILE), lambda i: (0, i)),
            pl.BlockSpec((1, _TILE), lambda i: (0, i)),
        ],
        out_specs=[
            pl.BlockSpec((1, _TILE), lambda i: (0, i)),
            pl.BlockSpec((1, 128), lambda i: (0, 0)),
        ],
        out_shape=[
            jax.ShapeDtypeStruct((1, _VOCAB), jnp.float32),
            jax.ShapeDtypeStruct((1, 128), jnp.float32),
        ],
        scratch_shapes=[
            pltpu.VMEM((1, _HIDDEN), jnp.float32),
            pltpu.VMEM((1, 1), jnp.float32),
            pltpu.VMEM((1, 1), jnp.float32),
        ],
        interpret=interpret,
    )(embeds, W1, b1.reshape(1, _HIDDEN), W2, b2.reshape(1, _VOCAB))

    log_probs = pl.pallas_call(
        _k2_body,
        grid=(_NT2,),
        in_specs=[
            pl.BlockSpec((1, _TILE2), lambda i: (0, i)),
            pl.BlockSpec((1, 128), lambda i: (0, 0)),
        ],
        out_specs=pl.BlockSpec((1, _TILE2), lambda i: (0, i)),
        out_shape=jax.ShapeDtypeStruct((1, _VOCAB), jnp.float32),
        interpret=interpret,
    )(out, lse)
    return log_probs


def kernel(inputs, emb, W1, b1, W2, b2):
    rows = _sc_gather(emb, inputs.astype(jnp.int32))
    embeds = rows.reshape(1, _CTX * _EMB_DIM)
    return _mlp(embeds, W1, b1, W2, b2)


# speedup vs baseline: 1.1853x; 1.1853x over previous
import functools

import jax
import jax.numpy as jnp
from jax import lax
from jax.experimental import pallas as pl
from jax.experimental.pallas import tpu as pltpu
from jax.experimental.pallas import tpu_sc as plsc

_VOCAB = 100000
_EMB_DIM = 200
_CTX = 2
_HIDDEN = 128

_TILE = 8192
_NT = (_VOCAB + _TILE - 1) // _TILE


def _sc_gather(emb, idx):
    mesh = plsc.VectorSubcoreMesh(core_axis_name="c", subcore_axis_name="s")

    @functools.partial(
        pl.kernel,
        mesh=mesh,
        out_type=jax.ShapeDtypeStruct((_CTX, _EMB_DIM), jnp.float32),
        scratch_types=[
            pltpu.VMEM((16,), jnp.int32),
            pltpu.VMEM((_CTX, _EMB_DIM), jnp.float32),
        ],
    )
    def k(emb_hbm, idx_hbm, out_hbm, idx_v, rows_v):
        wid = lax.axis_index("s") * 2 + lax.axis_index("c")

        @pl.when(wid == 0)
        def _():
            pltpu.sync_copy(idx_hbm, idx_v.at[pl.ds(0, _CTX)])
            lanes = idx_v[...]
            for r in range(_CTX):
                row = lanes[r]
                pltpu.sync_copy(emb_hbm.at[pl.ds(row, 1), :],
                                rows_v.at[pl.ds(r, 1), :])
            pltpu.sync_copy(rows_v, out_hbm)

    return k(emb, idx)


def _k1_body(emb_ref, w1_ref, b1_ref, w2_ref, b2_ref, out_ref, h_ref):
    i = pl.program_id(0)

    @pl.when(i == 0)
    def _():
        h = jnp.dot(emb_ref[...], w1_ref[...],
                    preferred_element_type=jnp.float32) + b1_ref[...]
        h_ref[...] = jnp.maximum(h, 0.0)

    out_ref[...] = jnp.dot(h_ref[...], w2_ref[...],
                           preferred_element_type=jnp.float32) + b2_ref[...]


def _k2_body(x_ref, o_ref):
    x = x_ref[...]
    mx = jnp.max(x)
    lse = mx + jnp.log(jnp.sum(jnp.exp(x - mx)))
    o_ref[...] = x - lse


def _mlp(embeds, W1, b1, W2, b2, interpret=False):
    out2d = pl.pallas_call(
        _k1_body,
        grid=(_NT,),
        in_specs=[
            pl.BlockSpec((1, _CTX * _EMB_DIM), lambda i: (0, 0)),
            pl.BlockSpec((_CTX * _EMB_DIM, _HIDDEN), lambda i: (0, 0)),
            pl.BlockSpec((1, _HIDDEN), lambda i: (0, 0)),
            pl.BlockSpec((_HIDDEN, _TILE), lambda i: (0, i)),
            pl.BlockSpec((1, _TILE), lambda i: (0, i)),
        ],
        out_specs=pl.BlockSpec((1, _TILE), lambda i: (0, i)),
        out_shape=jax.ShapeDtypeStruct((1, _VOCAB), jnp.float32),
        scratch_shapes=[
            pltpu.VMEM((1, _HIDDEN), jnp.float32),
        ],
        interpret=interpret,
    )(embeds, W1, b1.reshape(1, _HIDDEN), W2, b2.reshape(1, _VOCAB))

    log_probs = pl.pallas_call(
        _k2_body,
        out_shape=jax.ShapeDtypeStruct((1, _VOCAB), jnp.float32),
        interpret=interpret,
    )(out2d)
    return log_probs


def kernel(inputs, emb, W1, b1, W2, b2):
    rows = _sc_gather(emb, inputs.astype(jnp.int32))
    embeds = rows.reshape(1, _CTX * _EMB_DIM)
    return _mlp(embeds, W1, b1, W2, b2)


# speedup vs baseline: 1.2011x; 1.0134x over previous
import functools

import jax
import jax.numpy as jnp
from jax import lax
from jax.experimental import pallas as pl
from jax.experimental.pallas import tpu as pltpu
from jax.experimental.pallas import tpu_sc as plsc

_VOCAB = 100000
_EMB_DIM = 200
_CTX = 2
_HIDDEN = 128

_TILE = 8192
_NT = (_VOCAB + _TILE - 1) // _TILE


def _sc_gather(emb, idx):
    mesh = plsc.VectorSubcoreMesh(core_axis_name="c", subcore_axis_name="s")

    @functools.partial(
        pl.kernel,
        mesh=mesh,
        out_type=jax.ShapeDtypeStruct((_CTX, _EMB_DIM), jnp.float32),
        scratch_types=[
            pltpu.VMEM((16,), jnp.int32),
            pltpu.VMEM((_CTX, _EMB_DIM), jnp.float32),
        ],
    )
    def k(emb_hbm, idx_hbm, out_hbm, idx_v, rows_v):
        wid = lax.axis_index("s") * 2 + lax.axis_index("c")

        @pl.when(wid == 0)
        def _():
            pltpu.sync_copy(idx_hbm, idx_v.at[pl.ds(0, _CTX)])
            lanes = idx_v[...]
            for r in range(_CTX):
                row = lanes[r]
                pltpu.sync_copy(emb_hbm.at[pl.ds(row, 1), :],
                                rows_v.at[pl.ds(r, 1), :])
            pltpu.sync_copy(rows_v, out_hbm)

    return k(emb, idx)


_LAST = _NT - 1
_LAST_START = _LAST * _TILE
_LAST_W = _VOCAB - _LAST_START


def _k1_body(emb_ref, w1_ref, b1_ref, w2_ref, b2_ref, out_ref, h_ref):
    i = pl.program_id(0)

    @pl.when(i == 0)
    def _():
        h = jnp.dot(emb_ref[...], w1_ref[...],
                    preferred_element_type=jnp.float32) + b1_ref[...]
        h_ref[...] = jnp.maximum(h, 0.0)

    t = jnp.dot(h_ref[...], w2_ref[...],
                preferred_element_type=jnp.float32) + b2_ref[...]

    @pl.when(i < _LAST)
    def _():
        out_ref[:, pl.ds(i * _TILE, _TILE)] = t

    @pl.when(i == _LAST)
    def _():
        out_ref[:, _LAST_START:_VOCAB] = t[:, :_LAST_W]
        x = out_ref[...]
        mx = jnp.max(x)
        lse = mx + jnp.log(jnp.sum(jnp.exp(x - mx)))
        out_ref[...] = x - lse


def _mlp(embeds, W1, b1, W2, b2, interpret=False):
    return pl.pallas_call(
        _k1_body,
        grid=(_NT,),
        in_specs=[
            pl.BlockSpec((1, _CTX * _EMB_DIM), lambda i: (0, 0)),
            pl.BlockSpec((_CTX * _EMB_DIM, _HIDDEN), lambda i: (0, 0)),
            pl.BlockSpec((1, _HIDDEN), lambda i: (0, 0)),
            pl.BlockSpec((_HIDDEN, _TILE), lambda i: (0, i)),
            pl.BlockSpec((1, _TILE), lambda i: (0, i)),
        ],
        out_specs=pl.BlockSpec((1, _VOCAB), lambda i: (0, 0)),
        out_shape=jax.ShapeDtypeStruct((1, _VOCAB), jnp.float32),
        scratch_shapes=[
            pltpu.VMEM((1, _HIDDEN), jnp.float32),
        ],
        interpret=interpret,
    )(embeds, W1, b1.reshape(1, _HIDDEN), W2, b2.reshape(1, _VOCAB))


def kernel(inputs, emb, W1, b1, W2, b2):
    rows = _sc_gather(emb, inputs.astype(jnp.int32))
    embeds = rows.reshape(1, _CTX * _EMB_DIM)
    return _mlp(embeds, W1, b1, W2, b2)
